# Initial kernel scaffold; baseline (speedup 1.0000x reference)
import jax
import jax.numpy as jnp
from jax.experimental import pallas as pl


def kernel(x, adj, head, r1_g1, r1_g2, r1_b1, r1_b2, r2_g1, r2_g2, r2_b1, r2_b2, r1_m, r2_m, r1_w, r2_w):
    raise NotImplementedError("write your pallas kernel here")


# speedup vs baseline: 1.0583x; 1.0583x over previous
import functools

import jax
import jax.numpy as jnp
from jax.experimental import pallas as pl
from jax.experimental.pallas import tpu as pltpu

G_SIGMA = 1.0


def _lrelu(v):
    return jnp.where(v >= 0, v, 0.2 * v)


def _elu(v):
    return jnp.where(v > 0, v, jnp.exp(v) - 1.0)


def _layer_body(nrows, act, with_lsm,
                adj_ref, xfull_ref, wx_ref, wm_ref, m_ref, w_ref, fac_ref,
                *out_refs):
    i = pl.program_id(0)
    br = adj_ref.shape[0]
    f = xfull_ref.shape[1]
    xr = xfull_ref[pl.ds(i * br, br), :]
    mean = jnp.dot(adj_ref[...], xfull_ref[...],
                   preferred_element_type=jnp.float32)
    gb = (jnp.dot(xr, wx_ref[...], preferred_element_type=jnp.float32)
          + jnp.dot(mean, wm_ref[...], preferred_element_type=jnp.float32))
    gamma = _lrelu(gb[:, :f]) + 1.0
    beta = _lrelu(gb[:, f:])
    miss = xr + gamma * m_ref[...] + beta - mean
    h = mean + fac_ref[0] * miss
    out = jnp.dot(h, w_ref[...], preferred_element_type=jnp.float32)
    if act is not None:
        out = act(out)
    out_refs[0][...] = out
    out_refs[1][...] = miss
    if with_lsm:
        mx = jnp.max(out, axis=1, keepdims=True)
        sh = out - mx
        lse = jnp.log(jnp.sum(jnp.exp(sh), axis=1, keepdims=True))
        out_refs[2][...] = sh - lse


def _layer(xin, adj, wx, wm, m, w, fac, act, with_lsm):
    n, f = xin.shape
    fo = w.shape[1]
    br = next(b for b in (400, 200, 80, 16, 8, 1) if n % b == 0)
    grid = (n // br,)
    out_shapes = [
        jax.ShapeDtypeStruct((n, fo), jnp.float32),
        jax.ShapeDtypeStruct((n, f), jnp.float32),
    ]
    out_specs = [
        pl.BlockSpec((br, fo), lambda i: (i, 0)),
        pl.BlockSpec((br, f), lambda i: (i, 0)),
    ]
    if with_lsm:
        out_shapes.append(jax.ShapeDtypeStruct((n, fo), jnp.float32))
        out_specs.append(pl.BlockSpec((br, fo), lambda i: (i, 0)))
    return pl.pallas_call(
        functools.partial(_layer_body, br, act, with_lsm),
        grid=grid,
        in_specs=[
            pl.BlockSpec((br, n), lambda i: (i, 0)),
            pl.BlockSpec((n, f), lambda i: (0, 0)),
            pl.BlockSpec((f, 2 * f), lambda i: (0, 0)),
            pl.BlockSpec((f, 2 * f), lambda i: (0, 0)),
            pl.BlockSpec((1, f), lambda i: (0, 0)),
            pl.BlockSpec((f, fo), lambda i: (0, 0)),
            pl.BlockSpec(memory_space=pltpu.SMEM),
        ],
        out_specs=out_specs,
        out_shape=out_shapes,
        compiler_params=pltpu.CompilerParams(
            dimension_semantics=("parallel",),
            vmem_limit_bytes=100 * 1024 * 1024,
        ),
    )(adj, xin, wx, wm, m, w, fac)


def kernel(x, adj, head, r1_g1, r1_g2, r1_b1, r1_b2, r2_g1, r2_g2, r2_b1,
           r2_b2, r1_m, r2_m, r1_w, r2_w):
    fac = jnp.where(head != 0, 0.0, G_SIGMA).astype(jnp.float32).reshape(1)
    wx1 = jnp.concatenate([r1_g1, r1_b1], axis=1)
    wm1 = jnp.concatenate([r1_g2, r1_b2], axis=1)
    wx2 = jnp.concatenate([r2_g1, r2_b1], axis=1)
    wm2 = jnp.concatenate([r2_g2, r2_b2], axis=1)
    x1, out1 = _layer(x, adj, wx1, wm1, r1_m, r1_w, fac, _elu, False)
    x2, out2, lsm = _layer(x1, adj, wx2, wm2, r2_m, r2_w, fac, None, True)
    return x2, lsm, out1, out2
